# initial kernel scaffold (unmeasured)
import functools

import jax
import jax.numpy as jnp
from jax import lax
from jax.experimental import pallas as pl
from jax.experimental.pallas import tpu as pltpu

MESH = pl.DeviceIdType.MESH


def _matmul_body(a_ref, b_ref, o_ref, acc_ref, *, n_k):
    k = pl.program_id(2)

    @pl.when(k == 0)
    def _():
        acc_ref[...] = jnp.zeros_like(acc_ref)

    a = a_ref[...].astype(jnp.bfloat16)
    b = b_ref[...].astype(jnp.bfloat16)
    acc_ref[...] += lax.dot_general(
        a, b, (((1,), (1,)), ((), ())), preferred_element_type=jnp.float32
    )

    @pl.when(k == n_k - 1)
    def _():
        o_ref[...] = acc_ref[...].astype(jnp.bfloat16)


def _partial_matmul(dy, W, bm=2048, bn=2048, bk=512):
    m, kdim = dy.shape
    n = W.shape[0]
    grid = (m // bm, n // bn, kdim // bk)
    return pl.pallas_call(
        functools.partial(_matmul_body, n_k=grid[2]),
        grid=grid,
        in_specs=[
            pl.BlockSpec((bm, bk), lambda i, j, k: (i, k)),
            pl.BlockSpec((bn, bk), lambda i, j, k: (j, k)),
        ],
        out_specs=pl.BlockSpec((bm, bn), lambda i, j, k: (i, j)),
        out_shape=jax.ShapeDtypeStruct((m, n), jnp.bfloat16),
        scratch_shapes=[pltpu.VMEM((bm, bn), jnp.float32)],
    )(dy, W)


def _exchange_body(p_ref, r_ref, send_sem, recv_sem):
    my_x = lax.axis_index("x")
    my_y = lax.axis_index("y")
    nbr = (my_x, 1 - my_y)

    barrier = pltpu.get_barrier_semaphore()
    pl.semaphore_signal(barrier, inc=1, device_id=nbr, device_id_type=MESH)
    pl.semaphore_wait(barrier, 1)

    rdma = pltpu.make_async_remote_copy(
        src_ref=p_ref,
        dst_ref=r_ref,
        send_sem=send_sem,
        recv_sem=recv_sem,
        device_id=nbr,
        device_id_type=MESH,
    )
    rdma.start()
    rdma.wait()


def _exchange(p):
    return pl.pallas_call(
        _exchange_body,
        in_specs=[pl.BlockSpec(memory_space=pl.ANY)],
        out_specs=pl.BlockSpec(memory_space=pl.ANY),
        out_shape=jax.ShapeDtypeStruct(p.shape, p.dtype),
        scratch_shapes=[pltpu.SemaphoreType.DMA, pltpu.SemaphoreType.DMA],
        compiler_params=pltpu.CompilerParams(collective_id=0),
    )(p)


def _add_body(p_ref, r_ref, o_ref):
    o_ref[...] = p_ref[...].astype(jnp.float32) + r_ref[...].astype(jnp.float32)


def _add(p, r):
    m, n = p.shape
    bm = 512
    return pl.pallas_call(
        _add_body,
        grid=(m // bm,),
        in_specs=[
            pl.BlockSpec((bm, n), lambda i: (i, 0)),
            pl.BlockSpec((bm, n), lambda i: (i, 0)),
        ],
        out_specs=pl.BlockSpec((bm, n), lambda i: (i, 0)),
        out_shape=jax.ShapeDtypeStruct((m, n), jnp.float32),
    )(p, r)


def kernel(dy, W):
    p = _partial_matmul(dy, W)
    r = _exchange(p)
    return _add(p, r)


# baseline (device time: 730532 ns/iter reference)
import functools

import jax
import jax.numpy as jnp
from jax import lax
from jax.experimental import pallas as pl
from jax.experimental.pallas import tpu as pltpu

MESH = pl.DeviceIdType.MESH


def _matmul_body(a_ref, b_ref, o_ref, acc_ref, *, n_k):
    k = pl.program_id(2)

    @pl.when(k == 0)
    def _():
        acc_ref[...] = jnp.zeros_like(acc_ref)

    a = a_ref[...].astype(jnp.bfloat16)
    b = b_ref[...].astype(jnp.bfloat16)
    acc_ref[...] += lax.dot_general(
        a, b, (((1,), (1,)), ((), ())), preferred_element_type=jnp.float32
    )

    @pl.when(k == n_k - 1)
    def _():
        o_ref[...] = acc_ref[...].astype(jnp.bfloat16)


def _partial_matmul(dy, W, bm=2048, bn=2048, bk=512):
    m, kdim = dy.shape
    n = W.shape[0]
    grid = (m // bm, n // bn, kdim // bk)
    return pl.pallas_call(
        functools.partial(_matmul_body, n_k=grid[2]),
        grid=grid,
        in_specs=[
            pl.BlockSpec((bm, bk), lambda i, j, k: (i, k)),
            pl.BlockSpec((bn, bk), lambda i, j, k: (j, k)),
        ],
        out_specs=pl.BlockSpec((bm, bn), lambda i, j, k: (i, j)),
        out_shape=jax.ShapeDtypeStruct((m, n), jnp.bfloat16),
        scratch_shapes=[pltpu.VMEM((bm, bn), jnp.float32)],
        compiler_params=pltpu.CompilerParams(
            dimension_semantics=("parallel", "parallel", "arbitrary"),
            vmem_limit_bytes=100 * 1024 * 1024,
        ),
    )(dy, W)


def _exchange_body(p_ref, r_ref, send_sem, recv_sem):
    my_x = lax.axis_index("x")
    my_y = lax.axis_index("y")
    nbr = (my_x, 1 - my_y)

    barrier = pltpu.get_barrier_semaphore()
    pl.semaphore_signal(barrier, inc=1, device_id=nbr, device_id_type=MESH)
    pl.semaphore_wait(barrier, 1)

    rdma = pltpu.make_async_remote_copy(
        src_ref=p_ref,
        dst_ref=r_ref,
        send_sem=send_sem,
        recv_sem=recv_sem,
        device_id=nbr,
        device_id_type=MESH,
    )
    rdma.start()
    rdma.wait()


def _exchange(p):
    return pl.pallas_call(
        _exchange_body,
        in_specs=[pl.BlockSpec(memory_space=pl.ANY)],
        out_specs=pl.BlockSpec(memory_space=pl.ANY),
        out_shape=jax.ShapeDtypeStruct(p.shape, p.dtype),
        scratch_shapes=[pltpu.SemaphoreType.DMA, pltpu.SemaphoreType.DMA],
        compiler_params=pltpu.CompilerParams(collective_id=0),
    )(p)


def _add_body(p_ref, r_ref, o_ref):
    o_ref[...] = p_ref[...].astype(jnp.float32) + r_ref[...].astype(jnp.float32)


def _add(p, r):
    m, n = p.shape
    bm = 512
    return pl.pallas_call(
        _add_body,
        grid=(m // bm,),
        in_specs=[
            pl.BlockSpec((bm, n), lambda i: (i, 0)),
            pl.BlockSpec((bm, n), lambda i: (i, 0)),
        ],
        out_specs=pl.BlockSpec((bm, n), lambda i: (i, 0)),
        out_shape=jax.ShapeDtypeStruct((m, n), jnp.float32),
    )(p, r)


def kernel(dy, W):
    p = _partial_matmul(dy, W)
    r = _exchange(p)
    return _add(p, r)


# device time: 494285 ns/iter; 1.4780x vs baseline; 1.4780x over previous
import functools

import jax
import jax.numpy as jnp
from jax import lax
from jax.experimental import pallas as pl
from jax.experimental.pallas import tpu as pltpu

MESH = pl.DeviceIdType.MESH
HALF = 2048


def _matmul_body(idx_ref, a_ref, b_ref, o_ref, acc_ref, *, n_k):
    k = pl.program_id(1)

    @pl.when(k == 0)
    def _():
        acc_ref[...] = jnp.zeros_like(acc_ref)

    a = a_ref[...].astype(jnp.bfloat16)
    b = b_ref[...].astype(jnp.bfloat16)
    acc_ref[...] += lax.dot_general(
        a, b, (((1,), (1,)), ((), ())), preferred_element_type=jnp.float32
    )

    @pl.when(k == n_k - 1)
    def _():
        o_ref[0] = acc_ref[...].astype(jnp.bfloat16)


def _col_of(j, idx_ref):
    my_y = idx_ref[1]
    return jnp.where(j == 0, 1 - my_y, my_y)


def _partial_matmul(dy, W, my_x, my_y, bk=512):
    m, kdim = dy.shape
    n_k = kdim // bk
    grid = (2, n_k)
    idx = jnp.stack([my_x.astype(jnp.int32), my_y.astype(jnp.int32)])
    grid_spec = pltpu.PrefetchScalarGridSpec(
        num_scalar_prefetch=1,
        grid=grid,
        in_specs=[
            pl.BlockSpec((HALF, bk), lambda j, k, idx_ref: (idx_ref[0], k)),
            pl.BlockSpec((HALF, bk), lambda j, k, idx_ref: (_col_of(j, idx_ref), k)),
        ],
        out_specs=pl.BlockSpec(
            (1, HALF, HALF), lambda j, k, idx_ref: (_col_of(j, idx_ref), 0, 0)
        ),
        scratch_shapes=[pltpu.VMEM((HALF, HALF), jnp.float32)],
    )
    return pl.pallas_call(
        functools.partial(_matmul_body, n_k=n_k),
        grid_spec=grid_spec,
        out_shape=jax.ShapeDtypeStruct((2, HALF, HALF), jnp.bfloat16),
        compiler_params=pltpu.CompilerParams(
            dimension_semantics=("arbitrary", "arbitrary"),
            vmem_limit_bytes=100 * 1024 * 1024,
        ),
    )(idx, dy, W)


def _comm_body(p_ref, out_ref, r1_ref, send_sems, recv_sems):
    my_x = lax.axis_index("x")
    my_y = lax.axis_index("y")
    y_nbr = (my_x, 1 - my_y)
    x_nbr = (1 - my_x, my_y)
    s_own = 2 * my_x + my_y
    s_y = 2 * my_x + (1 - my_y)
    s_x = 2 * (1 - my_x) + my_y
    s_d = 2 * (1 - my_x) + (1 - my_y)

    barrier = pltpu.get_barrier_semaphore()
    pl.semaphore_signal(barrier, inc=1, device_id=y_nbr, device_id_type=MESH)
    pl.semaphore_signal(barrier, inc=1, device_id=x_nbr, device_id_type=MESH)
    pl.semaphore_wait(barrier, 2)

    ph1 = pltpu.make_async_remote_copy(
        src_ref=p_ref.at[1 - my_y],
        dst_ref=r1_ref,
        send_sem=send_sems.at[0],
        recv_sem=recv_sems.at[0],
        device_id=y_nbr,
        device_id_type=MESH,
    )
    ph1.start()
    ph1.wait()

    own = (
        p_ref[my_y].astype(jnp.float32) + r1_ref[...].astype(jnp.float32)
    ).astype(jnp.bfloat16)
    out_ref[s_own] = own

    ph2y = pltpu.make_async_remote_copy(
        src_ref=out_ref.at[s_own],
        dst_ref=out_ref.at[s_own],
        send_sem=send_sems.at[1],
        recv_sem=recv_sems.at[1],
        device_id=y_nbr,
        device_id_type=MESH,
    )
    ph2x = pltpu.make_async_remote_copy(
        src_ref=out_ref.at[s_own],
        dst_ref=out_ref.at[s_own],
        send_sem=send_sems.at[2],
        recv_sem=recv_sems.at[2],
        device_id=x_nbr,
        device_id_type=MESH,
    )
    ph2y.start()
    ph2x.start()

    ph2y.wait()
    ph3 = pltpu.make_async_remote_copy(
        src_ref=out_ref.at[s_y],
        dst_ref=out_ref.at[s_y],
        send_sem=send_sems.at[3],
        recv_sem=recv_sems.at[3],
        device_id=x_nbr,
        device_id_type=MESH,
    )
    ph3.start()
    ph2x.wait()
    ph3.wait()


def _allreduce_blocks(p):
    return pl.pallas_call(
        _comm_body,
        in_specs=[pl.BlockSpec(memory_space=pltpu.VMEM)],
        out_specs=pl.BlockSpec(memory_space=pltpu.VMEM),
        out_shape=jax.ShapeDtypeStruct((4, HALF, HALF), jnp.bfloat16),
        scratch_shapes=[
            pltpu.VMEM((HALF, HALF), jnp.bfloat16),
            pltpu.SemaphoreType.DMA((4,)),
            pltpu.SemaphoreType.DMA((4,)),
        ],
        compiler_params=pltpu.CompilerParams(
            collective_id=0,
            vmem_limit_bytes=100 * 1024 * 1024,
        ),
    )(p)


def _assemble_body(s_ref, o_ref):
    o_ref[...] = s_ref[0].astype(jnp.float32)


def _assemble(stacked):
    return pl.pallas_call(
        _assemble_body,
        grid=(2, 2),
        in_specs=[pl.BlockSpec((1, HALF, HALF), lambda i, j: (i * 2 + j, 0, 0))],
        out_specs=pl.BlockSpec((HALF, HALF), lambda i, j: (i, j)),
        out_shape=jax.ShapeDtypeStruct((2 * HALF, 2 * HALF), jnp.float32),
        compiler_params=pltpu.CompilerParams(
            dimension_semantics=("parallel", "parallel"),
            vmem_limit_bytes=100 * 1024 * 1024,
        ),
    )(stacked)


def kernel(dy, W):
    my_x = lax.axis_index("x")
    my_y = lax.axis_index("y")
    p = _partial_matmul(dy, W, my_x, my_y)
    stacked = _allreduce_blocks(p)
    return _assemble(stacked)


# device time: 387483 ns/iter; 1.8853x vs baseline; 1.2756x over previous
import functools

import jax
import jax.numpy as jnp
from jax import lax
from jax.experimental import pallas as pl
from jax.experimental.pallas import tpu as pltpu

MESH = pl.DeviceIdType.MESH
HALF = 2048
R = 4
CH = HALF // R
BK = 512
NK = 8192 // BK

PH1, PH2Y, PH2X, PH3A, PH3B = range(5)


def _col_of(j, idx_ref):
    my_y = idx_ref[1]
    return jnp.where(j == 0, 1 - my_y, my_y)


def _fused_body(idx_ref, a_ref, b_ref, out_ref, acc_ref, ship_ref,
                ssems, rsems):
    j = pl.program_id(0)
    r = pl.program_id(1)
    k = pl.program_id(2)
    my_x = idx_ref[0]
    my_y = idx_ref[1]
    y_nbr = (my_x, 1 - my_y)
    x_nbr = (1 - my_x, my_y)
    s_own = 2 * my_x + my_y
    s_y = 2 * my_x + (1 - my_y)
    s_x = 2 * (1 - my_x) + my_y
    s_d = 2 * (1 - my_x) + (1 - my_y)

    def out_chunk(slot, rr):
        return out_ref.at[pl.ds(slot * HALF + rr * CH, CH)]

    def ph1(rr):
        return pltpu.make_async_remote_copy(
            src_ref=ship_ref.at[pl.ds(rr * CH, CH)],
            dst_ref=out_chunk(s_y, rr),
            send_sem=ssems.at[PH1, rr], recv_sem=rsems.at[PH1, rr],
            device_id=y_nbr, device_id_type=MESH)

    def ph2y(rr):
        return pltpu.make_async_remote_copy(
            src_ref=out_chunk(s_own, rr), dst_ref=out_chunk(s_own, rr),
            send_sem=ssems.at[PH2Y, rr], recv_sem=rsems.at[PH2Y, rr],
            device_id=y_nbr, device_id_type=MESH)

    def ph2x(rr):
        return pltpu.make_async_remote_copy(
            src_ref=out_chunk(s_own, rr), dst_ref=out_chunk(s_own, rr),
            send_sem=ssems.at[PH2X, rr], recv_sem=rsems.at[PH2X, rr],
            device_id=x_nbr, device_id_type=MESH)

    def ph3a(rr):
        return pltpu.make_async_remote_copy(
            src_ref=out_chunk(s_y, rr), dst_ref=out_chunk(s_y, rr),
            send_sem=ssems.at[PH3A, rr], recv_sem=rsems.at[PH3A, rr],
            device_id=x_nbr, device_id_type=MESH)

    def ph3b(rr):
        return pltpu.make_async_remote_copy(
            src_ref=out_chunk(s_x, rr), dst_ref=out_chunk(s_x, rr),
            send_sem=ssems.at[PH3B, rr], recv_sem=rsems.at[PH3B, rr],
            device_id=y_nbr, device_id_type=MESH)

    def diag_recv(row, rr):
        return pltpu.make_async_remote_copy(
            src_ref=out_chunk(s_d, rr), dst_ref=out_chunk(s_d, rr),
            send_sem=ssems.at[row, rr], recv_sem=rsems.at[row, rr],
            device_id=x_nbr, device_id_type=MESH)

    @pl.when((j == 0) & (r == 0) & (k == 0))
    def _():
        barrier = pltpu.get_barrier_semaphore()
        pl.semaphore_signal(barrier, inc=1, device_id=y_nbr,
                            device_id_type=MESH)
        pl.semaphore_signal(barrier, inc=1, device_id=x_nbr,
                            device_id_type=MESH)
        pl.semaphore_wait(barrier, 2)

    @pl.when(k == 0)
    def _():
        acc_ref[...] = jnp.zeros_like(acc_ref)

    a = a_ref[...].astype(jnp.bfloat16)
    b = b_ref[...].astype(jnp.bfloat16)
    acc_ref[...] += lax.dot_general(
        a, b, (((1,), (1,)), ((), ())), preferred_element_type=jnp.float32
    )

    @pl.when((k == NK - 1) & (j == 0))
    def _():
        ship_ref[pl.ds(r * CH, CH)] = acc_ref[...].astype(jnp.bfloat16)
        ph1(r).start()

    @pl.when((k == NK - 1) & (j == 1))
    def _():
        ph1(r).wait()
        partner = out_ref[pl.ds(s_own * HALF + r * CH, CH)].astype(jnp.float32)
        own = (acc_ref[...] + partner).astype(jnp.bfloat16)
        out_ref[pl.ds(s_own * HALF + r * CH, CH)] = own
        ph2y(r).start()
        ph2x(r).start()

    @pl.when((k == NK - 1) & (j == 1) & (r == R - 1))
    def _():
        for rr in range(R):
            if rr % 2 == 0:
                ph2y(rr).wait_recv()
                ph3a(rr).start()
            else:
                ph2x(rr).wait_recv()
                ph3b(rr).start()
        for rr in range(R):
            if rr % 2 == 0:
                ph2x(rr).wait_recv()
                diag_recv(PH3A, rr).wait_recv()
            else:
                ph2y(rr).wait_recv()
                diag_recv(PH3B, rr).wait_recv()
        for rr in range(R):
            ph2y(rr).wait_send()
            ph2x(rr).wait_send()
            (ph3a(rr) if rr % 2 == 0 else ph3b(rr)).wait_send()


def _fused(dy, W, my_x, my_y):
    idx = jnp.stack([my_x.astype(jnp.int32), my_y.astype(jnp.int32)])
    grid_spec = pltpu.PrefetchScalarGridSpec(
        num_scalar_prefetch=1,
        grid=(2, R, NK),
        in_specs=[
            pl.BlockSpec((CH, BK), lambda j, r, k, idx_ref: (idx_ref[0] * R + r, k)),
            pl.BlockSpec((HALF, BK), lambda j, r, k, idx_ref: (_col_of(j, idx_ref), k)),
        ],
        out_specs=pl.BlockSpec((4 * HALF, HALF), lambda j, r, k, idx_ref: (0, 0)),
        scratch_shapes=[
            pltpu.VMEM((CH, HALF), jnp.float32),
            pltpu.VMEM((HALF, HALF), jnp.bfloat16),
            pltpu.SemaphoreType.DMA((5, R)),
            pltpu.SemaphoreType.DMA((5, R)),
        ],
    )
    return pl.pallas_call(
        _fused_body,
        grid_spec=grid_spec,
        out_shape=jax.ShapeDtypeStruct((4 * HALF, HALF), jnp.bfloat16),
        compiler_params=pltpu.CompilerParams(
            collective_id=0,
            dimension_semantics=("arbitrary", "arbitrary", "arbitrary"),
            vmem_limit_bytes=100 * 1024 * 1024,
        ),
    )(idx, dy, W)


def _assemble_body(s_ref, o_ref):
    o_ref[...] = s_ref[...].astype(jnp.float32)


def _assemble(flat):
    return pl.pallas_call(
        _assemble_body,
        grid=(2, 2),
        in_specs=[pl.BlockSpec((HALF, HALF), lambda i, j: (i * 2 + j, 0))],
        out_specs=pl.BlockSpec((HALF, HALF), lambda i, j: (i, j)),
        out_shape=jax.ShapeDtypeStruct((2 * HALF, 2 * HALF), jnp.float32),
        compiler_params=pltpu.CompilerParams(
            dimension_semantics=("parallel", "parallel"),
            vmem_limit_bytes=100 * 1024 * 1024,
        ),
    )(flat)


def kernel(dy, W):
    my_x = lax.axis_index("x")
    my_y = lax.axis_index("y")
    flat = _fused(dy, W, my_x, my_y)
    return _assemble(flat)


# device time: 319381 ns/iter; 2.2873x vs baseline; 1.2132x over previous
import functools

import jax
import jax.numpy as jnp
from jax import lax
from jax.experimental import pallas as pl
from jax.experimental.pallas import tpu as pltpu

MESH = pl.DeviceIdType.MESH
HALF = 2048
R = 2
CH = HALF // R
BK = 512
NK = 8192 // BK

PH1, PH2Y, PH2X, PH3A, PH3B = range(5)


def _col_of(j, idx_ref):
    my_y = idx_ref[1]
    return jnp.where(j == 0, 1 - my_y, my_y)


def _fused_body(idx_ref, a_ref, b_ref, out_ref, acc_ref, ship_ref,
                ssems, rsems):
    j = pl.program_id(0)
    r = pl.program_id(1)
    k = pl.program_id(2)
    my_x = idx_ref[0]
    my_y = idx_ref[1]
    y_nbr = (my_x, 1 - my_y)
    x_nbr = (1 - my_x, my_y)
    s_own = 2 * my_x + my_y
    s_y = 2 * my_x + (1 - my_y)
    s_x = 2 * (1 - my_x) + my_y
    s_d = 2 * (1 - my_x) + (1 - my_y)

    def out_chunk(slot, rr):
        return out_ref.at[pl.ds(slot * HALF + rr * CH, CH)]

    def ph1(rr):
        return pltpu.make_async_remote_copy(
            src_ref=ship_ref.at[pl.ds(rr * CH, CH)],
            dst_ref=out_chunk(s_y, rr),
            send_sem=ssems.at[PH1, rr], recv_sem=rsems.at[PH1, rr],
            device_id=y_nbr, device_id_type=MESH)

    def ph2y(rr):
        return pltpu.make_async_remote_copy(
            src_ref=out_chunk(s_own, rr), dst_ref=out_chunk(s_own, rr),
            send_sem=ssems.at[PH2Y, rr], recv_sem=rsems.at[PH2Y, rr],
            device_id=y_nbr, device_id_type=MESH)

    def ph2x(rr):
        return pltpu.make_async_remote_copy(
            src_ref=out_chunk(s_own, rr), dst_ref=out_chunk(s_own, rr),
            send_sem=ssems.at[PH2X, rr], recv_sem=rsems.at[PH2X, rr],
            device_id=x_nbr, device_id_type=MESH)

    def ph3a(rr):
        return pltpu.make_async_remote_copy(
            src_ref=out_chunk(s_y, rr), dst_ref=out_chunk(s_y, rr),
            send_sem=ssems.at[PH3A, rr], recv_sem=rsems.at[PH3A, rr],
            device_id=x_nbr, device_id_type=MESH)

    def ph3b(rr):
        return pltpu.make_async_remote_copy(
            src_ref=out_chunk(s_x, rr), dst_ref=out_chunk(s_x, rr),
            send_sem=ssems.at[PH3B, rr], recv_sem=rsems.at[PH3B, rr],
            device_id=y_nbr, device_id_type=MESH)

    def diag_recv(row, rr):
        return pltpu.make_async_remote_copy(
            src_ref=out_chunk(s_d, rr), dst_ref=out_chunk(s_d, rr),
            send_sem=ssems.at[row, rr], recv_sem=rsems.at[row, rr],
            device_id=x_nbr, device_id_type=MESH)

    @pl.when((j == 0) & (r == 0) & (k == 0))
    def _():
        barrier = pltpu.get_barrier_semaphore()
        pl.semaphore_signal(barrier, inc=1, device_id=y_nbr,
                            device_id_type=MESH)
        pl.semaphore_signal(barrier, inc=1, device_id=x_nbr,
                            device_id_type=MESH)
        pl.semaphore_wait(barrier, 2)

    @pl.when(k == 0)
    def _():
        acc_ref[...] = jnp.zeros_like(acc_ref)

    a = a_ref[...].astype(jnp.bfloat16)
    b = b_ref[...].astype(jnp.bfloat16)
    acc_ref[...] += lax.dot_general(
        a, b, (((1,), (1,)), ((), ())), preferred_element_type=jnp.float32
    )

    @pl.when((k == NK - 1) & (j == 0))
    def _():
        ship_ref[pl.ds(r * CH, CH)] = acc_ref[...].astype(jnp.bfloat16)
        ph1(r).start()

    @pl.when((k == NK - 1) & (j == 1))
    def _():
        ph1(r).wait()
        partner = out_ref[pl.ds(s_own * HALF + r * CH, CH)].astype(jnp.float32)
        own = (acc_ref[...] + partner).astype(jnp.bfloat16)
        out_ref[pl.ds(s_own * HALF + r * CH, CH)] = own
        ph2y(r).start()
        ph2x(r).start()

    @pl.when((k == NK - 1) & (j == 1) & (r == R - 1))
    def _():
        for rr in range(R):
            if rr % 2 == 0:
                ph2y(rr).wait_recv()
                ph3a(rr).start()
            else:
                ph2x(rr).wait_recv()
                ph3b(rr).start()
        for rr in range(R):
            if rr % 2 == 0:
                ph2x(rr).wait_recv()
                diag_recv(PH3A, rr).wait_recv()
            else:
                ph2y(rr).wait_recv()
                diag_recv(PH3B, rr).wait_recv()
        for rr in range(R):
            ph2y(rr).wait_send()
            ph2x(rr).wait_send()
            (ph3a(rr) if rr % 2 == 0 else ph3b(rr)).wait_send()


def _fused(dy, W, my_x, my_y):
    idx = jnp.stack([my_x.astype(jnp.int32), my_y.astype(jnp.int32)])
    grid_spec = pltpu.PrefetchScalarGridSpec(
        num_scalar_prefetch=1,
        grid=(2, R, NK),
        in_specs=[
            pl.BlockSpec((CH, BK), lambda j, r, k, idx_ref: (idx_ref[0] * R + r, k)),
            pl.BlockSpec((HALF, BK), lambda j, r, k, idx_ref: (_col_of(j, idx_ref), k)),
        ],
        out_specs=pl.BlockSpec((4 * HALF, HALF), lambda j, r, k, idx_ref: (0, 0)),
        scratch_shapes=[
            pltpu.VMEM((CH, HALF), jnp.float32),
            pltpu.VMEM((HALF, HALF), jnp.bfloat16),
            pltpu.SemaphoreType.DMA((5, R)),
            pltpu.SemaphoreType.DMA((5, R)),
        ],
    )
    return pl.pallas_call(
        _fused_body,
        grid_spec=grid_spec,
        out_shape=jax.ShapeDtypeStruct((4 * HALF, HALF), jnp.bfloat16),
        compiler_params=pltpu.CompilerParams(
            collective_id=0,
            dimension_semantics=("arbitrary", "arbitrary", "arbitrary"),
            vmem_limit_bytes=100 * 1024 * 1024,
        ),
    )(idx, dy, W)


def _assemble_body(s_ref, o_ref):
    o_ref[...] = s_ref[...].astype(jnp.float32)


def _assemble(flat):
    return pl.pallas_call(
        _assemble_body,
        grid=(2, 2),
        in_specs=[pl.BlockSpec((HALF, HALF), lambda i, j: (i * 2 + j, 0))],
        out_specs=pl.BlockSpec((HALF, HALF), lambda i, j: (i, j)),
        out_shape=jax.ShapeDtypeStruct((2 * HALF, 2 * HALF), jnp.float32),
        compiler_params=pltpu.CompilerParams(
            dimension_semantics=("parallel", "parallel"),
            vmem_limit_bytes=100 * 1024 * 1024,
        ),
    )(flat)


def kernel(dy, W):
    my_x = lax.axis_index("x")
    my_y = lax.axis_index("y")
    flat = _fused(dy, W, my_x, my_y)
    return _assemble(flat)
